# baseline (speedup 1.0000x reference)
import jax
import jax.numpy as jnp
from jax.experimental import pallas as pl

_BM = 256


def _gate_kernel(x_ref, w_ref, o_ref):
    o_ref[...] = jax.lax.dot_general(
        x_ref[...], w_ref[...], (((1,), (1,)), ((), ())),
        precision=jax.lax.Precision.DEFAULT,
        preferred_element_type=jnp.float32)


def kernel(hidden_states, weight):
    m, k = hidden_states.shape
    e = weight.shape[0]
    return pl.pallas_call(
        _gate_kernel,
        grid=(m // _BM,),
        in_specs=[
            pl.BlockSpec((_BM, k), lambda i: (i, 0)),
            pl.BlockSpec((e, k), lambda i: (0, 0)),
        ],
        out_specs=pl.BlockSpec((_BM, e), lambda i: (i, 0)),
        out_shape=jax.ShapeDtypeStruct((m, e), jnp.float32),
    )(hidden_states, weight)


# speedup vs baseline: 1.1937x; 1.1937x over previous
import jax
import jax.numpy as jnp
from jax.experimental import pallas as pl

_NS = 4
_BM = 128


def _gate_kernel(*refs):
    w_ref = refs[_NS]
    o_ref = refs[_NS + 1]
    for q in range(_NS):
        o_ref[q, :, :] = jax.lax.dot_general(
            refs[q][...], w_ref[...], (((1,), (1,)), ((), ())),
            precision=jax.lax.Precision.DEFAULT,
            preferred_element_type=jnp.float32)


def kernel(hidden_states, weight):
    m, k = hidden_states.shape
    e = weight.shape[0]
    mq = m // _NS
    nblk = mq // _BM
    in_specs = [
        pl.BlockSpec((_BM, k), lambda i, q=q: (q * nblk + i, 0))
        for q in range(_NS)
    ] + [pl.BlockSpec((e, k), lambda i: (0, 0))]
    out = pl.pallas_call(
        _gate_kernel,
        grid=(nblk,),
        in_specs=in_specs,
        out_specs=pl.BlockSpec((_NS, _BM, e), lambda i: (0, i, 0)),
        out_shape=jax.ShapeDtypeStruct((_NS, mq, e), jnp.float32),
    )(*([hidden_states] * _NS), weight)
    return out.reshape(m, e)
